# Initial kernel scaffold; baseline (speedup 1.0000x reference)
import jax
import jax.numpy as jnp
from jax.experimental import pallas as pl


def kernel(weight, target):
    raise NotImplementedError("write your pallas kernel here")


# speedup vs baseline: 45.6273x; 45.6273x over previous
import functools

import jax
import jax.numpy as jnp
from jax import lax
from jax.experimental import pallas as pl
from jax.experimental.pallas import tpu as pltpu
from jax.experimental.pallas import tpu_sc as plsc

_GAMMA = 0.05
_D = 512
_C = 10
_B = 16384

_NC = 2
_NS = 16
_L = 16
_NW = _NC * _NS
_PER_W = _B // _NW


def _hist_body(t_hbm, out_hbm, t_v, acc_v):
  wid = lax.axis_index("s") * _NC + lax.axis_index("c")
  pltpu.sync_copy(t_hbm.at[pl.ds(wid * _PER_W, _PER_W)], t_v)

  zero = jnp.zeros((_L,), jnp.int32)
  one = jnp.ones((_L,), jnp.int32)
  accs = [zero for _ in range(_C)]
  for i in range(_PER_W // _L):
    t = t_v[pl.ds(i * _L, _L)]
    for c in range(_C):
      accs[c] = accs[c] + jnp.where(t == c, one, zero)
  for c in range(_C):
    acc_v[c, :] = accs[c].astype(jnp.float32)

  pltpu.sync_copy(acc_v, out_hbm.at[wid])


@functools.cache
def _hist():
  return functools.partial(
      pl.kernel,
      mesh=plsc.VectorSubcoreMesh(
          core_axis_name="c", subcore_axis_name="s",
          num_cores=_NC, num_subcores=_NS),
      out_type=jax.ShapeDtypeStruct((_NW, _C, _L), jnp.float32),
      scratch_types=[
          pltpu.VMEM((_PER_W,), jnp.int32),
          pltpu.VMEM((_C, _L), jnp.float32),
      ],
  )(_hist_body)


def _loss_body(w_ref, p_ref, out_ref):
  w = w_ref[...]
  g = lax.dot_general(w, w, (((0,), (0,)), ((), ())),
                      preferred_element_type=jnp.float32)
  ii = lax.broadcasted_iota(jnp.int32, (_C, _C), 0)
  jj = lax.broadcasted_iota(jnp.int32, (_C, _C), 1)
  e = jnp.where(ii == jj, 0.0, jnp.exp(g * _GAMMA))
  lse = jnp.log(jnp.sum(e, axis=0, keepdims=True)) * (1.0 / _GAMMA)

  p = p_ref[...]
  rowsum = jnp.sum(p, axis=1, keepdims=True)
  rcls = lax.broadcasted_iota(jnp.int32, (_NW * _C, _C), 0) % _C
  ccls = lax.broadcasted_iota(jnp.int32, (_NW * _C, _C), 1)
  onehot = jnp.where(rcls == ccls, 1.0, 0.0)
  counts = jnp.sum(onehot * rowsum, axis=0, keepdims=True)

  out_ref[...] = jnp.sum(counts * lse, axis=1, keepdims=True) * (1.0 / _B)


_loss = pl.pallas_call(
    _loss_body,
    out_shape=jax.ShapeDtypeStruct((1, 1), jnp.float32),
)


@jax.jit
def kernel(weight, target):
  partials = _hist()(target.astype(jnp.int32))
  out = _loss(weight, partials.reshape(_NW * _C, _L))
  return out[0, 0]


# speedup vs baseline: 47.6410x; 1.0441x over previous
import functools

import jax
import jax.numpy as jnp
from jax import lax
from jax.experimental import pallas as pl
from jax.experimental.pallas import tpu as pltpu
from jax.experimental.pallas import tpu_sc as plsc

_GAMMA = 0.05
_D = 512
_C = 10
_B = 16384

_NC = 2
_NS = 16
_L = 16
_NW = _NC * _NS
_PER_W = _B // _NW


def _hist_body(t_hbm, out_hbm, t_v, acc_v):
  wid = lax.axis_index("s") * _NC + lax.axis_index("c")
  pltpu.sync_copy(t_hbm.at[pl.ds(wid * _PER_W, _PER_W)], t_v)

  zero = jnp.zeros((_L,), jnp.int32)
  one = jnp.ones((_L,), jnp.int32)

  def step(i, accs):
    t = t_v[pl.ds(i * _L, _L)]
    return tuple(a + jnp.where(t == c, one, zero) for c, a in enumerate(accs))

  accs = lax.fori_loop(0, _PER_W // _L, step, (zero,) * _C)
  for c in range(_C):
    acc_v[c, :] = accs[c].astype(jnp.float32)

  pltpu.sync_copy(acc_v, out_hbm.at[wid])


@functools.cache
def _hist():
  return functools.partial(
      pl.kernel,
      mesh=plsc.VectorSubcoreMesh(
          core_axis_name="c", subcore_axis_name="s",
          num_cores=_NC, num_subcores=_NS),
      out_type=jax.ShapeDtypeStruct((_NW, _C, _L), jnp.float32),
      scratch_types=[
          pltpu.VMEM((_PER_W,), jnp.int32),
          pltpu.VMEM((_C, _L), jnp.float32),
      ],
  )(_hist_body)


def _loss_body(w_ref, p_ref, out_ref):
  w = w_ref[...]
  g = lax.dot_general(w, w, (((0,), (0,)), ((), ())),
                      preferred_element_type=jnp.float32)
  ii = lax.broadcasted_iota(jnp.int32, (_C, _C), 0)
  jj = lax.broadcasted_iota(jnp.int32, (_C, _C), 1)
  e = jnp.where(ii == jj, 0.0, jnp.exp(g * _GAMMA))
  lse = jnp.log(jnp.sum(e, axis=0, keepdims=True)) * (1.0 / _GAMMA)

  p = p_ref[...]
  rowsum = jnp.sum(p, axis=1, keepdims=True)
  rcls = lax.broadcasted_iota(jnp.int32, (_NW * _C, _C), 0) % _C
  ccls = lax.broadcasted_iota(jnp.int32, (_NW * _C, _C), 1)
  onehot = jnp.where(rcls == ccls, 1.0, 0.0)
  counts = jnp.sum(onehot * rowsum, axis=0, keepdims=True)

  out_ref[...] = jnp.sum(counts * lse, axis=1, keepdims=True) * (1.0 / _B)


_loss = pl.pallas_call(
    _loss_body,
    out_shape=jax.ShapeDtypeStruct((1, 1), jnp.float32),
)


@jax.jit
def kernel(weight, target):
  partials = _hist()(target.astype(jnp.int32))
  out = _loss(weight, partials.reshape(_NW * _C, _L))
  return out[0, 0]


# speedup vs baseline: 47.6436x; 1.0001x over previous
import functools

import jax
import jax.numpy as jnp
from jax import lax
from jax.experimental import pallas as pl
from jax.experimental.pallas import tpu as pltpu
from jax.experimental.pallas import tpu_sc as plsc

_GAMMA = 0.05
_D = 512
_C = 10
_B = 16384

_NC = 2
_NS = 16
_L = 16
_NW = _NC * _NS
_PER_W = _B // _NW


def _hist_body(t_hbm, out_hbm, t_v, acc_v):
  wid = lax.axis_index("s") * _NC + lax.axis_index("c")
  pltpu.sync_copy(t_hbm.at[pl.ds(wid * _PER_W, _PER_W)], t_v)

  zero = jnp.zeros((_L,), jnp.int32)
  one = jnp.ones((_L,), jnp.int32)

  def step(i, accs):
    t = t_v[pl.ds(i * _L, _L)]
    return tuple(a + jnp.where(t == c, one, zero) for c, a in enumerate(accs))

  accs = lax.fori_loop(0, _PER_W // _L, step, (zero,) * _C)
  for c in range(_C):
    acc_v[c, :] = accs[c].astype(jnp.float32)

  pltpu.sync_copy(acc_v, out_hbm.at[wid])


@functools.cache
def _hist():
  return functools.partial(
      pl.kernel,
      mesh=plsc.VectorSubcoreMesh(
          core_axis_name="c", subcore_axis_name="s",
          num_cores=_NC, num_subcores=_NS),
      out_type=jax.ShapeDtypeStruct((_NW, _C, _L), jnp.float32),
      scratch_types=[
          pltpu.VMEM((_PER_W,), jnp.int32),
          pltpu.VMEM((_C, _L), jnp.float32),
      ],
  )(_hist_body)


def _lse_body(w_ref, lse_ref):
  w = w_ref[...]
  g = lax.dot_general(w, w, (((0,), (0,)), ((), ())),
                      preferred_element_type=jnp.float32)
  ii = lax.broadcasted_iota(jnp.int32, (_C, _C), 0)
  jj = lax.broadcasted_iota(jnp.int32, (_C, _C), 1)
  e = jnp.where(ii == jj, 0.0, jnp.exp(g * _GAMMA))
  lse_ref[...] = jnp.log(jnp.sum(e, axis=0, keepdims=True)) * (1.0 / _GAMMA)


_lse = pl.pallas_call(
    _lse_body,
    out_shape=jax.ShapeDtypeStruct((1, _C), jnp.float32),
)


def _combine_body(lse_ref, p_ref, out_ref):
  lse = lse_ref[...]
  p = p_ref[...]
  rowsum = jnp.sum(p, axis=1, keepdims=True)
  rcls = lax.broadcasted_iota(jnp.int32, (_NW * _C, _C), 0) % _C
  ccls = lax.broadcasted_iota(jnp.int32, (_NW * _C, _C), 1)
  onehot = jnp.where(rcls == ccls, 1.0, 0.0)
  counts = jnp.sum(onehot * rowsum, axis=0, keepdims=True)
  out_ref[...] = jnp.sum(counts * lse, axis=1, keepdims=True) * (1.0 / _B)


_combine = pl.pallas_call(
    _combine_body,
    out_shape=jax.ShapeDtypeStruct((1, 1), jnp.float32),
)


@jax.jit
def kernel(weight, target):
  partials = _hist()(target.astype(jnp.int32))
  lse = _lse(weight)
  out = _combine(lse, partials.reshape(_NW * _C, _L))
  return out[0, 0]


# speedup vs baseline: 51.1341x; 1.0733x over previous
import functools

import jax
import jax.numpy as jnp
from jax import lax
from jax.experimental import pallas as pl
from jax.experimental.pallas import tpu as pltpu
from jax.experimental.pallas import tpu_sc as plsc

_GAMMA = 0.05
_D = 512
_C = 10
_B = 16384

_NC = 2
_NS = 16
_L = 16
_NW = _NC * _NS
_PER_W = _B // _NW


def _hist_body(t_hbm, out_hbm, t_v, acc_v):
  wid = lax.axis_index("s") * _NC + lax.axis_index("c")
  pltpu.sync_copy(t_hbm.at[pl.ds(wid * _PER_W, _PER_W)], t_v)

  zero = jnp.zeros((_L,), jnp.int32)
  one = jnp.ones((_L,), jnp.int32)

  def step(i, accs):
    t = t_v[pl.ds(i * _L, _L)]
    return tuple(a + jnp.where(t == c, one, zero) for c, a in enumerate(accs))

  accs = lax.fori_loop(0, _PER_W // _L, step, (zero,) * _C)
  for c in range(_C):
    acc_v[c, :] = accs[c].astype(jnp.float32)

  pltpu.sync_copy(acc_v, out_hbm.at[wid])


@functools.cache
def _hist():
  return functools.partial(
      pl.kernel,
      mesh=plsc.VectorSubcoreMesh(
          core_axis_name="c", subcore_axis_name="s",
          num_cores=_NC, num_subcores=_NS),
      out_type=jax.ShapeDtypeStruct((_NW, _C, _L), jnp.float32),
      scratch_types=[
          pltpu.VMEM((_PER_W,), jnp.int32),
          pltpu.VMEM((_C, _L), jnp.float32),
      ],
  )(_hist_body)


def _lse_body(w_ref, lse_ref):
  w = w_ref[...]
  g = lax.dot_general(w, w, (((0,), (0,)), ((), ())),
                      preferred_element_type=jnp.float32)
  ii = lax.broadcasted_iota(jnp.int32, (_C, _C), 0)
  jj = lax.broadcasted_iota(jnp.int32, (_C, _C), 1)
  e = jnp.where(ii == jj, 0.0, jnp.exp(g * _GAMMA))
  lse_ref[...] = jnp.log(jnp.sum(e, axis=0, keepdims=True)) * (1.0 / _GAMMA)


_lse = pl.pallas_call(
    _lse_body,
    out_shape=jax.ShapeDtypeStruct((1, _C), jnp.float32),
)


def _combine_body(lse_ref, p_ref, out_ref):
  lse = lse_ref[...]
  p = p_ref[...]
  counts = jnp.sum(jnp.sum(p, axis=2), axis=0, keepdims=True)
  out_ref[...] = jnp.sum(counts * lse, axis=1, keepdims=True) * (1.0 / _B)


_combine = pl.pallas_call(
    _combine_body,
    out_shape=jax.ShapeDtypeStruct((1, 1), jnp.float32),
)


@jax.jit
def kernel(weight, target):
  partials = _hist()(target)
  lse = _lse(weight)
  out = _combine(lse, partials)
  return out[0, 0]


# speedup vs baseline: 54.5265x; 1.0663x over previous
import functools

import jax
import jax.numpy as jnp
from jax import lax
from jax.experimental import pallas as pl
from jax.experimental.pallas import tpu as pltpu
from jax.experimental.pallas import tpu_sc as plsc

_GAMMA = 0.05
_D = 512
_C = 10
_B = 16384

_NC = 1
_NS = 16
_L = 16
_NW = _NC * _NS
_PER_W = _B // _NW


def _hist_body(t_hbm, out_hbm, t_v, acc_v):
  wid = lax.axis_index("s") * _NC + lax.axis_index("c")
  pltpu.sync_copy(t_hbm.at[pl.ds(wid * _PER_W, _PER_W)], t_v)

  zero = jnp.zeros((_L,), jnp.int32)
  one = jnp.ones((_L,), jnp.int32)

  def step(i, accs):
    t = t_v[pl.ds(i * _L, _L)]
    return tuple(a + jnp.where(t == c, one, zero) for c, a in enumerate(accs))

  accs = lax.fori_loop(0, _PER_W // _L, step, (zero,) * _C)
  for c in range(_C):
    acc_v[c, :] = accs[c].astype(jnp.float32)

  pltpu.sync_copy(acc_v, out_hbm.at[wid])


@functools.cache
def _hist():
  return functools.partial(
      pl.kernel,
      mesh=plsc.VectorSubcoreMesh(
          core_axis_name="c", subcore_axis_name="s",
          num_cores=_NC, num_subcores=_NS),
      out_type=jax.ShapeDtypeStruct((_NW, _C, _L), jnp.float32),
      scratch_types=[
          pltpu.VMEM((_PER_W,), jnp.int32),
          pltpu.VMEM((_C, _L), jnp.float32),
      ],
  )(_hist_body)


def _lse_body(w_ref, lse_ref):
  w = w_ref[...]
  g = lax.dot_general(w, w, (((0,), (0,)), ((), ())),
                      preferred_element_type=jnp.float32)
  ii = lax.broadcasted_iota(jnp.int32, (_C, _C), 0)
  jj = lax.broadcasted_iota(jnp.int32, (_C, _C), 1)
  e = jnp.where(ii == jj, 0.0, jnp.exp(g * _GAMMA))
  lse_ref[...] = jnp.log(jnp.sum(e, axis=0, keepdims=True)) * (1.0 / _GAMMA)


_lse = pl.pallas_call(
    _lse_body,
    out_shape=jax.ShapeDtypeStruct((1, _C), jnp.float32),
)


def _combine_body(lse_ref, p_ref, out_ref):
  lse = lse_ref[...]
  p = p_ref[...]
  counts = jnp.sum(jnp.sum(p, axis=2), axis=0, keepdims=True)
  out_ref[...] = jnp.sum(counts * lse, axis=1, keepdims=True) * (1.0 / _B)


_combine = pl.pallas_call(
    _combine_body,
    out_shape=jax.ShapeDtypeStruct((1, 1), jnp.float32),
)


@jax.jit
def kernel(weight, target):
  partials = _hist()(target)
  lse = _lse(weight)
  out = _combine(lse, partials)
  return out[0, 0]
